# baseline (speedup 1.0000x reference)
import functools

import jax
import jax.numpy as jnp
from jax.experimental import pallas as pl
from jax.experimental.pallas import tpu as pltpu

_NB = 4096
_NBUF = 4


def _w_copy(W_ref, wbuf, wsem, slot, base, rows):
    return pltpu.make_async_copy(
        W_ref.at[pl.ds(base, rows), :],
        wbuf.at[slot, pl.ds(0, rows), :] if rows != _NB else wbuf.at[slot],
        wsem.at[slot],
    )


def _fc_kernel(x_ref, b_ref, W_ref, out_ref, wbuf, wsem, *, V, n, rem):
    x = x_ref[:]

    for k in range(_NBUF):
        _w_copy(W_ref, wbuf, wsem, k, k * _NB, _NB).start()

    def step(i, carry):
        m, s = carry
        slot = jax.lax.rem(i, _NBUF)
        _w_copy(W_ref, wbuf, wsem, slot, i * _NB, _NB).wait()
        logits = jax.lax.dot_general(
            x, wbuf[slot],
            dimension_numbers=(((1,), (1,)), ((), ())),
            preferred_element_type=jnp.float32,
        ) + b_ref[:, pl.ds(i * _NB, _NB)]
        out_ref[:, pl.ds(i * _NB, _NB)] = logits

        m_blk = jnp.max(logits, axis=1, keepdims=True)
        m_new = jnp.maximum(m, m_blk)
        s_new = s * jnp.exp(m - m_new) + jnp.sum(
            jnp.exp(logits - m_new), axis=1, keepdims=True)

        nxt = i + _NBUF
        nslot = jax.lax.rem(nxt, _NBUF)

        @pl.when(nxt < n - 1)
        def _():
            _w_copy(W_ref, wbuf, wsem, nslot, nxt * _NB, _NB).start()

        @pl.when(nxt == n - 1)
        def _():
            _w_copy(W_ref, wbuf, wsem, nslot, nxt * _NB, rem).start()

        return m_new, s_new

    m0 = jnp.full((x.shape[0], 1), -jnp.inf, dtype=jnp.float32)
    s0 = jnp.zeros((x.shape[0], 1), dtype=jnp.float32)
    m, s = jax.lax.fori_loop(0, n - 1, step, (m0, s0))

    lslot = (n - 1) % _NBUF
    _w_copy(W_ref, wbuf, wsem, lslot, (n - 1) * _NB, rem).wait()
    logits = jax.lax.dot_general(
        x, wbuf[lslot, :rem, :],
        dimension_numbers=(((1,), (1,)), ((), ())),
        preferred_element_type=jnp.float32,
    ) + b_ref[:, pl.ds((n - 1) * _NB, rem)]
    out_ref[:, pl.ds((n - 1) * _NB, rem)] = logits
    m_blk = jnp.max(logits, axis=1, keepdims=True)
    m_new = jnp.maximum(m, m_blk)
    s = s * jnp.exp(m - m_new) + jnp.sum(
        jnp.exp(logits - m_new), axis=1, keepdims=True)
    lse = m_new + jnp.log(s)

    out_ref[:, :] = out_ref[:, :] - lse


@jax.jit
def kernel(x, W, b):
    B, K = x.shape
    V = W.shape[0]
    n = pl.cdiv(V, _NB)
    rem = V - (n - 1) * _NB
    b2 = b.reshape(1, V)

    return pl.pallas_call(
        functools.partial(_fc_kernel, V=V, n=n, rem=rem),
        in_specs=[
            pl.BlockSpec(memory_space=pltpu.VMEM),
            pl.BlockSpec(memory_space=pltpu.VMEM),
            pl.BlockSpec(memory_space=pl.ANY),
        ],
        out_specs=pl.BlockSpec(memory_space=pltpu.VMEM),
        out_shape=jax.ShapeDtypeStruct((B, V), jnp.float32),
        scratch_shapes=[
            pltpu.VMEM((_NBUF, _NB, K), jnp.float32),
            pltpu.SemaphoreType.DMA((_NBUF,)),
        ],
    )(x, b2, W)


# speedup vs baseline: 1.0066x; 1.0066x over previous
import functools

import jax
import jax.numpy as jnp
from jax.experimental import pallas as pl
from jax.experimental.pallas import tpu as pltpu

_NB = 4096
_NBUF = 4


def _w_copy(W_ref, wbuf, wsem, slot, base, rows):
    return pltpu.make_async_copy(
        W_ref.at[pl.ds(base, rows), :],
        wbuf.at[slot, pl.ds(0, rows), :] if rows != _NB else wbuf.at[slot],
        wsem.at[slot],
    )


_NO = 16384


def _fc_kernel(x_ref, b_ref, W_ref, out_hbm, wbuf, out_ref, wsem, osem,
               *, V, n, rem):
    x = x_ref[:]

    for k in range(_NBUF):
        _w_copy(W_ref, wbuf, wsem, k, k * _NB, _NB).start()

    def step(i, carry):
        m, s = carry
        slot = jax.lax.rem(i, _NBUF)
        _w_copy(W_ref, wbuf, wsem, slot, i * _NB, _NB).wait()
        logits = jax.lax.dot_general(
            x, wbuf[slot],
            dimension_numbers=(((1,), (1,)), ((), ())),
            preferred_element_type=jnp.float32,
        ) + b_ref[:, pl.ds(i * _NB, _NB)]
        out_ref[:, pl.ds(i * _NB, _NB)] = logits

        m_blk = jnp.max(logits, axis=1, keepdims=True)
        m_new = jnp.maximum(m, m_blk)
        s_new = s * jnp.exp(m - m_new) + jnp.sum(
            jnp.exp(logits - m_new), axis=1, keepdims=True)

        nxt = i + _NBUF
        nslot = jax.lax.rem(nxt, _NBUF)

        @pl.when(nxt < n - 1)
        def _():
            _w_copy(W_ref, wbuf, wsem, nslot, nxt * _NB, _NB).start()

        @pl.when(nxt == n - 1)
        def _():
            _w_copy(W_ref, wbuf, wsem, nslot, nxt * _NB, rem).start()

        return m_new, s_new

    m0 = jnp.full((x.shape[0], 1), -jnp.inf, dtype=jnp.float32)
    s0 = jnp.zeros((x.shape[0], 1), dtype=jnp.float32)
    m, s = jax.lax.fori_loop(0, n - 1, step, (m0, s0))

    lslot = (n - 1) % _NBUF
    _w_copy(W_ref, wbuf, wsem, lslot, (n - 1) * _NB, rem).wait()
    logits = jax.lax.dot_general(
        x, wbuf[lslot, :rem, :],
        dimension_numbers=(((1,), (1,)), ((), ())),
        preferred_element_type=jnp.float32,
    ) + b_ref[:, pl.ds((n - 1) * _NB, rem)]
    out_ref[:, pl.ds((n - 1) * _NB, rem)] = logits
    m_blk = jnp.max(logits, axis=1, keepdims=True)
    m_new = jnp.maximum(m, m_blk)
    s = s * jnp.exp(m - m_new) + jnp.sum(
        jnp.exp(logits - m_new), axis=1, keepdims=True)
    lse = m_new + jnp.log(s)

    no = (V // _NO) * _NO
    tail = V - no
    copies = []
    if tail:
        out_ref[:, pl.ds(no, tail)] = out_ref[:, pl.ds(no, tail)] - lse
        cp = pltpu.make_async_copy(
            out_ref.at[:, pl.ds(no, tail)],
            out_hbm.at[:, pl.ds(no, tail)],
            osem.at[V // _NO],
        )
        cp.start()
        copies.append(cp)
    for k in range(V // _NO):
        out_ref[:, pl.ds(k * _NO, _NO)] = (
            out_ref[:, pl.ds(k * _NO, _NO)] - lse)
        cp = pltpu.make_async_copy(
            out_ref.at[:, pl.ds(k * _NO, _NO)],
            out_hbm.at[:, pl.ds(k * _NO, _NO)],
            osem.at[k],
        )
        cp.start()
        copies.append(cp)
    for cp in copies:
        cp.wait()


@jax.jit
def kernel(x, W, b):
    B, K = x.shape
    V = W.shape[0]
    n = pl.cdiv(V, _NB)
    rem = V - (n - 1) * _NB
    b2 = b.reshape(1, V)

    return pl.pallas_call(
        functools.partial(_fc_kernel, V=V, n=n, rem=rem),
        in_specs=[
            pl.BlockSpec(memory_space=pltpu.VMEM),
            pl.BlockSpec(memory_space=pltpu.VMEM),
            pl.BlockSpec(memory_space=pl.ANY),
        ],
        out_specs=pl.BlockSpec(memory_space=pl.ANY),
        out_shape=jax.ShapeDtypeStruct((B, V), jnp.float32),
        scratch_shapes=[
            pltpu.VMEM((_NBUF, _NB, K), jnp.float32),
            pltpu.VMEM((B, V), jnp.float32),
            pltpu.SemaphoreType.DMA((_NBUF,)),
            pltpu.SemaphoreType.DMA((pl.cdiv(V, _NO),)),
        ],
    )(x, b2, W)


# speedup vs baseline: 1.0098x; 1.0032x over previous
import functools

import jax
import jax.numpy as jnp
from jax.experimental import pallas as pl
from jax.experimental.pallas import tpu as pltpu

_NB = 4096
_NBUF = 4


def _w_copy(W_ref, wbuf, wsem, slot, base, rows):
    return pltpu.make_async_copy(
        W_ref.at[pl.ds(base, rows), :],
        wbuf.at[slot, pl.ds(0, rows), :] if rows != _NB else wbuf.at[slot],
        wsem.at[slot],
    )


_NO = 16384


def _fc_kernel(x_ref, b_ref, W_ref, out_hbm, wbuf, out_ref, wsem, osem,
               *, V, n, rem):
    x = x_ref[:]

    for k in range(_NBUF):
        _w_copy(W_ref, wbuf, wsem, k, k * _NB, _NB).start()

    def step(i, carry):
        m, s = carry
        slot = jax.lax.rem(i, _NBUF)
        _w_copy(W_ref, wbuf, wsem, slot, i * _NB, _NB).wait()
        logits = jax.lax.dot_general(
            x.astype(jnp.bfloat16), wbuf[slot].astype(jnp.bfloat16),
            dimension_numbers=(((1,), (1,)), ((), ())),
            preferred_element_type=jnp.float32,
        ) + b_ref[:, pl.ds(i * _NB, _NB)]
        out_ref[:, pl.ds(i * _NB, _NB)] = logits

        m_blk = jnp.max(logits, axis=1, keepdims=True)
        m_new = jnp.maximum(m, m_blk)
        s_new = s * jnp.exp(m - m_new) + jnp.sum(
            jnp.exp(logits - m_new), axis=1, keepdims=True)

        nxt = i + _NBUF
        nslot = jax.lax.rem(nxt, _NBUF)

        @pl.when(nxt < n - 1)
        def _():
            _w_copy(W_ref, wbuf, wsem, nslot, nxt * _NB, _NB).start()

        @pl.when(nxt == n - 1)
        def _():
            _w_copy(W_ref, wbuf, wsem, nslot, nxt * _NB, rem).start()

        return m_new, s_new

    m0 = jnp.full((x.shape[0], 1), -jnp.inf, dtype=jnp.float32)
    s0 = jnp.zeros((x.shape[0], 1), dtype=jnp.float32)
    m, s = jax.lax.fori_loop(0, n - 1, step, (m0, s0))

    lslot = (n - 1) % _NBUF
    _w_copy(W_ref, wbuf, wsem, lslot, (n - 1) * _NB, rem).wait()
    logits = jax.lax.dot_general(
        x.astype(jnp.bfloat16), wbuf[lslot, :rem, :].astype(jnp.bfloat16),
        dimension_numbers=(((1,), (1,)), ((), ())),
        preferred_element_type=jnp.float32,
    ) + b_ref[:, pl.ds((n - 1) * _NB, rem)]
    out_ref[:, pl.ds((n - 1) * _NB, rem)] = logits
    m_blk = jnp.max(logits, axis=1, keepdims=True)
    m_new = jnp.maximum(m, m_blk)
    s = s * jnp.exp(m - m_new) + jnp.sum(
        jnp.exp(logits - m_new), axis=1, keepdims=True)
    lse = m_new + jnp.log(s)

    no = (V // _NO) * _NO
    tail = V - no
    copies = []
    if tail:
        out_ref[:, pl.ds(no, tail)] = out_ref[:, pl.ds(no, tail)] - lse
        cp = pltpu.make_async_copy(
            out_ref.at[:, pl.ds(no, tail)],
            out_hbm.at[:, pl.ds(no, tail)],
            osem.at[V // _NO],
        )
        cp.start()
        copies.append(cp)
    for k in range(V // _NO):
        out_ref[:, pl.ds(k * _NO, _NO)] = (
            out_ref[:, pl.ds(k * _NO, _NO)] - lse)
        cp = pltpu.make_async_copy(
            out_ref.at[:, pl.ds(k * _NO, _NO)],
            out_hbm.at[:, pl.ds(k * _NO, _NO)],
            osem.at[k],
        )
        cp.start()
        copies.append(cp)
    for cp in copies:
        cp.wait()


@jax.jit
def kernel(x, W, b):
    B, K = x.shape
    V = W.shape[0]
    n = pl.cdiv(V, _NB)
    rem = V - (n - 1) * _NB
    b2 = b.reshape(1, V)

    return pl.pallas_call(
        functools.partial(_fc_kernel, V=V, n=n, rem=rem),
        in_specs=[
            pl.BlockSpec(memory_space=pltpu.VMEM),
            pl.BlockSpec(memory_space=pltpu.VMEM),
            pl.BlockSpec(memory_space=pl.ANY),
        ],
        out_specs=pl.BlockSpec(memory_space=pl.ANY),
        out_shape=jax.ShapeDtypeStruct((B, V), jnp.float32),
        scratch_shapes=[
            pltpu.VMEM((_NBUF, _NB, K), jnp.float32),
            pltpu.VMEM((B, V), jnp.float32),
            pltpu.SemaphoreType.DMA((_NBUF,)),
            pltpu.SemaphoreType.DMA((pl.cdiv(V, _NO),)),
        ],
    )(x, b2, W)


# speedup vs baseline: 1.0147x; 1.0048x over previous
import functools

import jax
import jax.numpy as jnp
from jax.experimental import pallas as pl
from jax.experimental.pallas import tpu as pltpu

_NB = 4096
_NBUF = 4


def _w_copy(W_ref, wbuf, wsem, slot, base, rows):
    return pltpu.make_async_copy(
        W_ref.at[pl.ds(base, rows), :],
        wbuf.at[slot, pl.ds(0, rows), :] if rows != _NB else wbuf.at[slot],
        wsem.at[slot],
    )


_NO = 16384


def _fc_kernel(x_ref, b_ref, W_ref, out_hbm, wbuf, out_ref, wsem, osem,
               *, V, n, rem):
    x = x_ref[:]

    for k in range(_NBUF):
        _w_copy(W_ref, wbuf, wsem, k, k * _NB, _NB).start()

    def chunk(i, m, s):
        slot = jax.lax.rem(i, _NBUF)
        _w_copy(W_ref, wbuf, wsem, slot, i * _NB, _NB).wait()
        logits = jax.lax.dot_general(
            x, wbuf[slot],
            dimension_numbers=(((1,), (1,)), ((), ())),
            preferred_element_type=jnp.float32,
        ) + b_ref[:, pl.ds(i * _NB, _NB)]
        out_ref[:, pl.ds(i * _NB, _NB)] = logits

        m_blk = jnp.max(logits, axis=1, keepdims=True)
        m_new = jnp.maximum(m, m_blk)
        s_new = s * jnp.exp(m - m_new) + jnp.sum(
            jnp.exp(logits - m_new), axis=1, keepdims=True)

        nxt = i + _NBUF
        nslot = jax.lax.rem(nxt, _NBUF)

        @pl.when(nxt < n - 1)
        def _():
            _w_copy(W_ref, wbuf, wsem, nslot, nxt * _NB, _NB).start()

        @pl.when(nxt == n - 1)
        def _():
            _w_copy(W_ref, wbuf, wsem, nslot, nxt * _NB, rem).start()

        return m_new, s_new

    def step(p, carry):
        m, s = carry
        m, s = chunk(2 * p, m, s)
        m, s = chunk(2 * p + 1, m, s)
        return m, s

    m0 = jnp.full((x.shape[0], 1), -jnp.inf, dtype=jnp.float32)
    s0 = jnp.zeros((x.shape[0], 1), dtype=jnp.float32)
    npair = (n - 1) // 2
    m, s = jax.lax.fori_loop(0, npair, step, (m0, s0))
    for i in range(2 * npair, n - 1):
        m, s = chunk(i, m, s)

    lslot = (n - 1) % _NBUF
    _w_copy(W_ref, wbuf, wsem, lslot, (n - 1) * _NB, rem).wait()
    logits = jax.lax.dot_general(
        x, wbuf[lslot, :rem, :],
        dimension_numbers=(((1,), (1,)), ((), ())),
        preferred_element_type=jnp.float32,
    ) + b_ref[:, pl.ds((n - 1) * _NB, rem)]
    out_ref[:, pl.ds((n - 1) * _NB, rem)] = logits
    m_blk = jnp.max(logits, axis=1, keepdims=True)
    m_new = jnp.maximum(m, m_blk)
    s = s * jnp.exp(m - m_new) + jnp.sum(
        jnp.exp(logits - m_new), axis=1, keepdims=True)
    lse = m_new + jnp.log(s)

    no = (V // _NO) * _NO
    tail = V - no
    copies = []
    if tail:
        out_ref[:, pl.ds(no, tail)] = out_ref[:, pl.ds(no, tail)] - lse
        cp = pltpu.make_async_copy(
            out_ref.at[:, pl.ds(no, tail)],
            out_hbm.at[:, pl.ds(no, tail)],
            osem.at[V // _NO],
        )
        cp.start()
        copies.append(cp)
    for k in range(V // _NO):
        out_ref[:, pl.ds(k * _NO, _NO)] = (
            out_ref[:, pl.ds(k * _NO, _NO)] - lse)
        cp = pltpu.make_async_copy(
            out_ref.at[:, pl.ds(k * _NO, _NO)],
            out_hbm.at[:, pl.ds(k * _NO, _NO)],
            osem.at[k],
        )
        cp.start()
        copies.append(cp)
    for cp in copies:
        cp.wait()


@jax.jit
def kernel(x, W, b):
    B, K = x.shape
    V = W.shape[0]
    n = pl.cdiv(V, _NB)
    rem = V - (n - 1) * _NB
    b2 = b.reshape(1, V)

    return pl.pallas_call(
        functools.partial(_fc_kernel, V=V, n=n, rem=rem),
        in_specs=[
            pl.BlockSpec(memory_space=pltpu.VMEM),
            pl.BlockSpec(memory_space=pltpu.VMEM),
            pl.BlockSpec(memory_space=pl.ANY),
        ],
        out_specs=pl.BlockSpec(memory_space=pl.ANY),
        out_shape=jax.ShapeDtypeStruct((B, V), jnp.float32),
        scratch_shapes=[
            pltpu.VMEM((_NBUF, _NB, K), jnp.float32),
            pltpu.VMEM((B, V), jnp.float32),
            pltpu.SemaphoreType.DMA((_NBUF,)),
            pltpu.SemaphoreType.DMA((pl.cdiv(V, _NO),)),
        ],
    )(x, b2, W)


# speedup vs baseline: 1.0540x; 1.0387x over previous
import functools

import jax
import jax.numpy as jnp
from jax.experimental import pallas as pl
from jax.experimental.pallas import tpu as pltpu

_NB = 4096
_NBUF = 4


def _w_copy(W_ref, wbuf, wsem, slot, base, rows):
    return pltpu.make_async_copy(
        W_ref.at[pl.ds(base, rows), :],
        wbuf.at[slot, pl.ds(0, rows), :] if rows != _NB else wbuf.at[slot],
        wsem.at[slot],
    )


_NO = 16384


def _fc_kernel(x_ref, b_ref, W_ref, out_hbm, wbuf, out_ref, wsem, osem,
               *, V, n, rem):
    x = x_ref[:]

    for k in range(_NBUF):
        _w_copy(W_ref, wbuf, wsem, k, k * _NB, _NB).start()

    def chunk(i, m, s):
        slot = jax.lax.rem(i, _NBUF)
        _w_copy(W_ref, wbuf, wsem, slot, i * _NB, _NB).wait()
        logits = jax.lax.dot_general(
            x, wbuf[slot],
            dimension_numbers=(((1,), (1,)), ((), ())),
            preferred_element_type=jnp.float32,
        ) + b_ref[:, pl.ds(i * _NB, _NB)]
        out_ref[:, pl.ds(i * _NB, _NB)] = logits

        m_blk = jnp.max(logits, axis=1, keepdims=True)
        m_new = jnp.maximum(m, m_blk)
        s_new = s * jnp.exp(m - m_new) + jnp.sum(
            jnp.exp(logits - m_new), axis=1, keepdims=True)

        nxt = i + _NBUF
        nslot = jax.lax.rem(nxt, _NBUF)

        @pl.when(nxt < n - 1)
        def _():
            _w_copy(W_ref, wbuf, wsem, nslot, nxt * _NB, _NB).start()

        @pl.when(nxt == n - 1)
        def _():
            _w_copy(W_ref, wbuf, wsem, nslot, nxt * _NB, rem).start()

        return m_new, s_new

    m = jnp.full((x.shape[0], 1), -jnp.inf, dtype=jnp.float32)
    s = jnp.zeros((x.shape[0], 1), dtype=jnp.float32)
    for i in range(n - 1):
        m, s = chunk(i, m, s)

    lslot = (n - 1) % _NBUF
    _w_copy(W_ref, wbuf, wsem, lslot, (n - 1) * _NB, rem).wait()
    logits = jax.lax.dot_general(
        x, wbuf[lslot, :rem, :],
        dimension_numbers=(((1,), (1,)), ((), ())),
        preferred_element_type=jnp.float32,
    ) + b_ref[:, pl.ds((n - 1) * _NB, rem)]
    out_ref[:, pl.ds((n - 1) * _NB, rem)] = logits
    m_blk = jnp.max(logits, axis=1, keepdims=True)
    m_new = jnp.maximum(m, m_blk)
    s = s * jnp.exp(m - m_new) + jnp.sum(
        jnp.exp(logits - m_new), axis=1, keepdims=True)
    lse = m_new + jnp.log(s)

    no = (V // _NO) * _NO
    tail = V - no
    copies = []
    if tail:
        out_ref[:, pl.ds(no, tail)] = out_ref[:, pl.ds(no, tail)] - lse
        cp = pltpu.make_async_copy(
            out_ref.at[:, pl.ds(no, tail)],
            out_hbm.at[:, pl.ds(no, tail)],
            osem.at[V // _NO],
        )
        cp.start()
        copies.append(cp)
    for k in range(V // _NO):
        out_ref[:, pl.ds(k * _NO, _NO)] = (
            out_ref[:, pl.ds(k * _NO, _NO)] - lse)
        cp = pltpu.make_async_copy(
            out_ref.at[:, pl.ds(k * _NO, _NO)],
            out_hbm.at[:, pl.ds(k * _NO, _NO)],
            osem.at[k],
        )
        cp.start()
        copies.append(cp)
    for cp in copies:
        cp.wait()


@jax.jit
def kernel(x, W, b):
    B, K = x.shape
    V = W.shape[0]
    n = pl.cdiv(V, _NB)
    rem = V - (n - 1) * _NB
    b2 = b.reshape(1, V)

    return pl.pallas_call(
        functools.partial(_fc_kernel, V=V, n=n, rem=rem),
        in_specs=[
            pl.BlockSpec(memory_space=pltpu.VMEM),
            pl.BlockSpec(memory_space=pltpu.VMEM),
            pl.BlockSpec(memory_space=pl.ANY),
        ],
        out_specs=pl.BlockSpec(memory_space=pl.ANY),
        out_shape=jax.ShapeDtypeStruct((B, V), jnp.float32),
        scratch_shapes=[
            pltpu.VMEM((_NBUF, _NB, K), jnp.float32),
            pltpu.VMEM((B, V), jnp.float32),
            pltpu.SemaphoreType.DMA((_NBUF,)),
            pltpu.SemaphoreType.DMA((pl.cdiv(V, _NO),)),
        ],
    )(x, b2, W)
